# baseline (speedup 1.0000x reference)
import jax
import jax.numpy as jnp
from jax.experimental import pallas as pl
from jax.experimental.pallas import tpu as pltpu

_N_PAPER = 100000
_N_AUTHOR = 50000
_N_FIELD = 10000
_EMBED = 128


def _concat_copy_kernel(p_ref, a_ref, f_ref, o_ref, sem_p, sem_a, sem_f):
    cp = pltpu.make_async_copy(p_ref, o_ref.at[pl.ds(0, _N_PAPER)], sem_p)
    ca = pltpu.make_async_copy(a_ref, o_ref.at[pl.ds(_N_PAPER, _N_AUTHOR)], sem_a)
    cf = pltpu.make_async_copy(
        f_ref, o_ref.at[pl.ds(_N_PAPER + _N_AUTHOR, _N_FIELD)], sem_f)
    cp.start()
    ca.start()
    cf.start()
    cp.wait()
    ca.wait()
    cf.wait()


def kernel(embed_paper, embed_author, embed_field):
    total = _N_PAPER + _N_AUTHOR + _N_FIELD
    return pl.pallas_call(
        _concat_copy_kernel,
        out_shape=jax.ShapeDtypeStruct((total, _EMBED), jnp.float32),
        in_specs=[
            pl.BlockSpec(memory_space=pl.ANY),
            pl.BlockSpec(memory_space=pl.ANY),
            pl.BlockSpec(memory_space=pl.ANY),
        ],
        out_specs=pl.BlockSpec(memory_space=pl.ANY),
        scratch_shapes=[
            pltpu.SemaphoreType.DMA,
            pltpu.SemaphoreType.DMA,
            pltpu.SemaphoreType.DMA,
        ],
    )(embed_paper, embed_author, embed_field)


# speedup vs baseline: 1.0240x; 1.0240x over previous
import jax
import jax.numpy as jnp
from jax.experimental import pallas as pl
from jax.experimental.pallas import tpu as pltpu

_N_PAPER = 100000
_N_AUTHOR = 50000
_N_FIELD = 10000
_EMBED = 128
_CHUNK = 6250


def _concat_copy_kernel(p_ref, a_ref, f_ref, o_ref, sem):
    copies = []
    i = 0
    for src, n_rows, out_base in (
        (p_ref, _N_PAPER, 0),
        (a_ref, _N_AUTHOR, _N_PAPER),
        (f_ref, _N_FIELD, _N_PAPER + _N_AUTHOR),
    ):
        for c in range(n_rows // _CHUNK):
            copies.append(pltpu.make_async_copy(
                src.at[pl.ds(c * _CHUNK, _CHUNK)],
                o_ref.at[pl.ds(out_base + c * _CHUNK, _CHUNK)],
                sem.at[i]))
            i += 1
    for cp in copies:
        cp.start()
    for cp in copies:
        cp.wait()


def kernel(embed_paper, embed_author, embed_field):
    total = _N_PAPER + _N_AUTHOR + _N_FIELD
    n_dmas = total // _CHUNK
    return pl.pallas_call(
        _concat_copy_kernel,
        out_shape=jax.ShapeDtypeStruct((total, _EMBED), jnp.float32),
        in_specs=[
            pl.BlockSpec(memory_space=pl.ANY),
            pl.BlockSpec(memory_space=pl.ANY),
            pl.BlockSpec(memory_space=pl.ANY),
        ],
        out_specs=pl.BlockSpec(memory_space=pl.ANY),
        scratch_shapes=[pltpu.SemaphoreType.DMA((n_dmas,))],
    )(embed_paper, embed_author, embed_field)


# speedup vs baseline: 43.8204x; 42.7925x over previous
import jax
import jax.numpy as jnp
from jax.experimental import pallas as pl

_N_PAPER = 100000
_N_AUTHOR = 50000
_N_FIELD = 10000
_EMBED = 128
_CHUNK = 5000
_PB = _N_PAPER // _CHUNK
_AB = _N_AUTHOR // _CHUNK
_FB = _N_FIELD // _CHUNK


def _concat_kernel(p_ref, a_ref, f_ref, o_ref):
    i = pl.program_id(0)

    @pl.when(i < _PB)
    def _():
        o_ref[...] = p_ref[...]

    @pl.when(jnp.logical_and(i >= _PB, i < _PB + _AB))
    def _():
        o_ref[...] = a_ref[...]

    @pl.when(i >= _PB + _AB)
    def _():
        o_ref[...] = f_ref[...]


def kernel(embed_paper, embed_author, embed_field):
    total = _N_PAPER + _N_AUTHOR + _N_FIELD
    return pl.pallas_call(
        _concat_kernel,
        grid=(_PB + _AB + _FB,),
        out_shape=jax.ShapeDtypeStruct((total, _EMBED), jnp.float32),
        in_specs=[
            pl.BlockSpec((_CHUNK, _EMBED),
                         lambda i: (jnp.minimum(i, _PB - 1), 0)),
            pl.BlockSpec((_CHUNK, _EMBED),
                         lambda i: (jnp.clip(i - _PB, 0, _AB - 1), 0)),
            pl.BlockSpec((_CHUNK, _EMBED),
                         lambda i: (jnp.clip(i - _PB - _AB, 0, _FB - 1), 0)),
        ],
        out_specs=pl.BlockSpec((_CHUNK, _EMBED), lambda i: (i, 0)),
    )(embed_paper, embed_author, embed_field)


# speedup vs baseline: 46.8720x; 1.0696x over previous
import jax
import jax.numpy as jnp
from jax.experimental import pallas as pl

_N_PAPER = 100000
_N_AUTHOR = 50000
_N_FIELD = 10000
_EMBED = 128
_CHUNK = 10000
_PB = _N_PAPER // _CHUNK
_AB = _N_AUTHOR // _CHUNK
_FB = _N_FIELD // _CHUNK


def _concat_kernel(p_ref, a_ref, f_ref, o_ref):
    i = pl.program_id(0)

    @pl.when(i < _PB)
    def _():
        o_ref[...] = p_ref[...]

    @pl.when(jnp.logical_and(i >= _PB, i < _PB + _AB))
    def _():
        o_ref[...] = a_ref[...]

    @pl.when(i >= _PB + _AB)
    def _():
        o_ref[...] = f_ref[...]


def kernel(embed_paper, embed_author, embed_field):
    total = _N_PAPER + _N_AUTHOR + _N_FIELD
    return pl.pallas_call(
        _concat_kernel,
        grid=(_PB + _AB + _FB,),
        out_shape=jax.ShapeDtypeStruct((total, _EMBED), jnp.float32),
        in_specs=[
            pl.BlockSpec((_CHUNK, _EMBED),
                         lambda i: (jnp.minimum(i, _PB - 1), 0)),
            pl.BlockSpec((_CHUNK, _EMBED),
                         lambda i: (jnp.clip(i - _PB, 0, _AB - 1), 0)),
            pl.BlockSpec((_CHUNK, _EMBED),
                         lambda i: (jnp.clip(i - _PB - _AB, 0, _FB - 1), 0)),
        ],
        out_specs=pl.BlockSpec((_CHUNK, _EMBED), lambda i: (i, 0)),
    )(embed_paper, embed_author, embed_field)
